# Initial kernel scaffold; baseline (speedup 1.0000x reference)
import jax
import jax.numpy as jnp
from jax.experimental import pallas as pl


def kernel(x, edge_index, drug_index, label, W1_up, W1_down, W1_bias, g1_up, g1_down, W2_up, W2_down, W2_bias, g2_up, g2_down, W3_up, W3_down, W3_bias, g3_up, g3_down, P1, P2):
    raise NotImplementedError("write your pallas kernel here")


# speedup vs baseline: 5.2269x; 5.2269x over previous
import functools

import jax
import jax.numpy as jnp
from jax import lax
from jax.experimental import pallas as pl
from jax.experimental.pallas import tpu as pltpu
from jax.experimental.pallas import tpu_sc as plsc

N = 10000
E = 320000
NG = 300000
ND = E - NG
D = 128

NC = 2
NS = 16
CH = 128
NCHUNK = E // CH

_MESH = dict(core_axis_name="c", subcore_axis_name="s", num_cores=NC,
             num_subcores=NS)
_SC_PARAMS = pltpu.CompilerParams(needs_layout_passes=False)

_ROWS_LO = 624
_ROWS_HI = N - 15 * _ROWS_LO


def _sc_degree(ei3):
    mesh = plsc.VectorSubcoreMesh(**_MESH)
    jmax = (NCHUNK + NC * NS - 1) // (NC * NS)

    @functools.partial(
        pl.kernel,
        out_type=jax.ShapeDtypeStruct((NC * NS, 1, 2 * N), jnp.float32),
        mesh=mesh,
        compiler_params=_SC_PARAMS,
        scratch_types=[
            pltpu.VMEM((CH,), jnp.int32),
            pltpu.VMEM((CH,), jnp.int32),
            pltpu.VMEM((1, 2 * N), jnp.float32),
        ],
    )
    def k(ei_hbm, out_hbm, dbuf, sbuf, hist):
        c = lax.axis_index("c")
        s = lax.axis_index("s")
        wid = c * NS + s

        @pl.loop(0, 2 * N, step=16)
        def _(i):
            hist[0, pl.ds(i, 16)] = jnp.zeros((16,), jnp.float32)

        ones = jnp.ones((16,), jnp.float32)
        hrow = hist.at[0]

        @pl.loop(0, jmax)
        def _(j):
            cid = j * NC * NS + wid

            @pl.when(cid < NCHUNK)
            def _():
                eb = cid * CH
                pltpu.sync_copy(ei_hbm.at[1, 0, pl.ds(eb, CH)], dbuf)
                pltpu.sync_copy(ei_hbm.at[0, 0, pl.ds(eb, CH)], sbuf)
                for g in range(CH // 16):
                    sl = pl.ds(g * 16, 16)
                    plsc.addupdate_scatter(hrow, [dbuf[sl] * 2], ones)
                    plsc.addupdate_scatter(hrow, [sbuf[sl] * 2 + 1], ones)

        pltpu.sync_copy(hist, out_hbm.at[wid])

    return k(ei3)


def _tc_deg_finish_body(h_ref, o_ref):
    deg = jnp.sum(h_ref[...], axis=0)
    o_ref[...] = jnp.where(deg > 0, 1.0 / deg, 0.0)


def _tc_deg_finish(hists):
    return pl.pallas_call(
        _tc_deg_finish_body,
        out_shape=jax.ShapeDtypeStruct((1, 2 * N), jnp.float32),
    )(hists)


def _sc_prop(tables, ei3, gw3, zeros):
    mesh = plsc.VectorSubcoreMesh(**_MESH)
    jmax = (NCHUNK + NS - 1) // NS

    @functools.partial(
        pl.kernel,
        out_type=jax.ShapeDtypeStruct((2, N, D), jnp.float32),
        mesh=mesh,
        compiler_params=_SC_PARAMS,
        scratch_types=[
            pltpu.VMEM((CH,), jnp.int32),
            pltpu.VMEM((CH,), jnp.int32),
            pltpu.VMEM((CH,), jnp.float32),
            pltpu.VMEM((CH, D), jnp.float32),
            pltpu.VMEM_SHARED((N, D), jnp.float32),
            pltpu.SemaphoreType.DMA,
        ],
    )
    def k(tab_hbm, ei_hbm, gw_hbm, z_hbm, s_hbm, gidx, sidx, wbuf, rows,
          accum, sem):
        c = lax.axis_index("c")
        s = lax.axis_index("s")

        rb = s * _ROWS_LO

        @pl.when(s < NS - 1)
        def _():
            pltpu.sync_copy(z_hbm.at[pl.ds(rb, _ROWS_LO)],
                            accum.at[pl.ds(rb, _ROWS_LO)])

        @pl.when(s == NS - 1)
        def _():
            pltpu.sync_copy(z_hbm.at[pl.ds(rb, _ROWS_HI)],
                            accum.at[pl.ds(rb, _ROWS_HI)])

        plsc.subcore_barrier()

        tab_c = tab_hbm.at[c]

        @pl.loop(0, jmax)
        def _(j):
            cid = j * NS + s

            @pl.when(cid < NCHUNK)
            def _():
                eb = cid * CH
                pltpu.sync_copy(ei_hbm.at[c, 0, pl.ds(eb, CH)], gidx)
                pltpu.sync_copy(ei_hbm.at[1 - c, 0, pl.ds(eb, CH)], sidx)
                pltpu.sync_copy(gw_hbm.at[c, 0, pl.ds(eb, CH)], wbuf)
                pltpu.async_copy(tab_c.at[gidx], rows, sem).wait()

                @pl.loop(0, CH)
                def _(r):
                    wv = plsc.load_gather(
                        wbuf, [jnp.full((16,), r, jnp.int32)])
                    for q in range(D // 16):
                        sl = pl.ds(q * 16, 16)
                        rows[r, sl] = rows[r, sl] * wv

                pltpu.sync_copy(rows, accum.at[sidx], add=True)

        plsc.subcore_barrier()

        @pl.when(s < NS - 1)
        def _():
            pltpu.sync_copy(accum.at[pl.ds(rb, _ROWS_LO)],
                            s_hbm.at[c].at[pl.ds(rb, _ROWS_LO)])

        @pl.when(s == NS - 1)
        def _():
            pltpu.sync_copy(accum.at[pl.ds(rb, _ROWS_HI)],
                            s_hbm.at[c].at[pl.ds(rb, _ROWS_HI)])

    return k(tables, ei3, gw3, zeros)


def _sc_pair_gather(h, idx):
    mesh = plsc.VectorSubcoreMesh(**_MESH)
    B = idx.shape[0]
    per_tile = B // (NC * NS)

    @functools.partial(
        pl.kernel,
        out_type=jax.ShapeDtypeStruct((B, 3 * D), jnp.float32),
        mesh=mesh,
        compiler_params=_SC_PARAMS,
        scratch_types=[
            pltpu.VMEM((B,), jnp.int32),
            pltpu.VMEM((per_tile, 3 * D), jnp.float32),
            pltpu.SemaphoreType.DMA,
        ],
    )
    def k(h_hbm, i_hbm, o_hbm, idx_v, rows, sem):
        c = lax.axis_index("c")
        s = lax.axis_index("s")
        wid = c * NS + s
        base = wid * per_tile
        pltpu.sync_copy(i_hbm, idx_v)
        pltpu.async_copy(h_hbm.at[idx_v.at[pl.ds(base, per_tile)]], rows,
                         sem).wait()
        pltpu.sync_copy(rows, o_hbm.at[pl.ds(base, per_tile)])

    return k(h, idx)


_RB = 2000


def _tc_proj_body(h_ref, wu_ref, wd_ref, wb_ref, tab_ref, bias_ref):
    hb = h_ref[...]
    tab_ref[0] = jnp.dot(hb, wu_ref[...], preferred_element_type=jnp.float32)
    tab_ref[1] = jnp.dot(hb, wd_ref[...], preferred_element_type=jnp.float32)
    bias_ref[...] = jnp.dot(hb, wb_ref[...],
                            preferred_element_type=jnp.float32)


def _tc_proj(h, wu, wd, wb):
    din = h.shape[1]
    return pl.pallas_call(
        _tc_proj_body,
        grid=(N // _RB,),
        in_specs=[
            pl.BlockSpec((_RB, din), lambda i: (i, 0)),
            pl.BlockSpec((din, D), lambda i: (0, 0)),
            pl.BlockSpec((din, D), lambda i: (0, 0)),
            pl.BlockSpec((din, D), lambda i: (0, 0)),
        ],
        out_specs=[
            pl.BlockSpec((2, _RB, D), lambda i: (0, i, 0)),
            pl.BlockSpec((_RB, D), lambda i: (i, 0)),
        ],
        out_shape=[
            jax.ShapeDtypeStruct((2, N, D), jnp.float32),
            jax.ShapeDtypeStruct((N, D), jnp.float32),
        ],
    )(h, wu, wd, wb)


def _tc_finish_body(s_ref, dinv_ref, bias_ref, h_ref):
    d = dinv_ref[...]
    xu = s_ref[0] * d[:, 0:1]
    xd = s_ref[1] * d[:, 1:2]
    bx = bias_ref[...]
    ss = (jnp.sum(xu * xu, axis=1) + jnp.sum(xd * xd, axis=1)
          + jnp.sum(bx * bx, axis=1))
    inv = 1.0 / jnp.maximum(jnp.sqrt(ss), 1e-12)

    def leaky(t):
        return jnp.where(t >= 0, t, 0.1 * t)

    h_ref[:, 0:D] = leaky(xu * inv[:, None])
    h_ref[:, D:2 * D] = leaky(xd * inv[:, None])
    h_ref[:, 2 * D:3 * D] = leaky(bx * inv[:, None])


def _tc_finish(S, deg_inv, bias):
    return pl.pallas_call(
        _tc_finish_body,
        grid=(N // _RB,),
        in_specs=[
            pl.BlockSpec((2, _RB, D), lambda i: (0, i, 0)),
            pl.BlockSpec((_RB, 2), lambda i: (i, 0)),
            pl.BlockSpec((_RB, D), lambda i: (i, 0)),
        ],
        out_specs=pl.BlockSpec((_RB, 3 * D), lambda i: (i, 0)),
        out_shape=jax.ShapeDtypeStruct((N, 3 * D), jnp.float32),
    )(S, deg_inv, bias)


def _tc_decoder_body(a_ref, b_ref, p1_ref, p2_ref, o_ref):
    a = a_ref[...]
    p1 = p1_ref[...]
    t1 = jnp.dot(a, p1, preferred_element_type=jnp.float32)
    t2 = jnp.dot(t1, p2_ref[...], preferred_element_type=jnp.float32)
    p = lax.dot_general(t2, p1, (((1,), (1,)), ((), ())),
                        preferred_element_type=jnp.float32)
    o_ref[...] = jnp.sum(p * b_ref[...], axis=1, keepdims=True)


def _tc_decoder(a, b, p1, p2):
    npairs = a.shape[0]
    return pl.pallas_call(
        _tc_decoder_body,
        out_shape=jax.ShapeDtypeStruct((npairs, 1), jnp.float32),
    )(a, b, p1, p2)


def kernel(x, edge_index, drug_index, label, W1_up, W1_down, W1_bias, g1_up,
           g1_down, W2_up, W2_down, W2_bias, g2_up, g2_down, W3_up, W3_down,
           W3_bias, g3_up, g3_down, P1, P2):
    ei3 = edge_index.astype(jnp.int32).reshape(2, 1, E)
    zeros = jnp.zeros((N, D), jnp.float32)
    ones_d = jnp.ones((ND,), jnp.float32)

    hists = _sc_degree(ei3)
    deg_inv = _tc_deg_finish(hists).reshape(N, 2)

    h = x
    for wu, wd, wb, gu, gd in (
        (W1_up, W1_down, W1_bias, g1_up, g1_down),
        (W2_up, W2_down, W2_bias, g2_up, g2_down),
        (W3_up, W3_down, W3_bias, g3_up, g3_down),
    ):
        tabs, bias = _tc_proj(h, wu, wd, wb)
        gw3 = jnp.stack([jnp.concatenate([gu, ones_d]),
                         jnp.concatenate([gd, ones_d])]).reshape(2, 1, E)
        S = _sc_prop(tabs, ei3, gw3, zeros)
        h = _tc_finish(S, deg_inv, bias)

    di = drug_index.reshape(-1, 2).astype(jnp.int32)
    ia = di[:, 0] - 1
    ib = di[:, 1] - 1
    ia = jnp.where(ia < 0, ia + N, ia)
    ib = jnp.where(ib < 0, ib + N, ib)
    ab = _sc_pair_gather(h, jnp.concatenate([ia, ib]))
    a = ab[: di.shape[0]]
    b = ab[di.shape[0]:]
    return _tc_decoder(a, b, P1, P2)


# speedup vs baseline: 6.0796x; 1.1631x over previous
import functools

import jax
import jax.numpy as jnp
from jax import lax
from jax.experimental import pallas as pl
from jax.experimental.pallas import tpu as pltpu
from jax.experimental.pallas import tpu_sc as plsc

N = 10000
E = 320000
NG = 300000
ND = E - NG
D = 128

NC = 2
NS = 16
CH = 128
NCHP = 2560
EP = NCHP * CH
PAD = EP - E
NCH_T = NCHP // NS
NCH_D = NCHP // (NC * NS)

_MESH = dict(core_axis_name="c", subcore_axis_name="s", num_cores=NC,
             num_subcores=NS)
_SC_PARAMS = pltpu.CompilerParams(needs_layout_passes=False)

_ROWS_LO = 624
_ROWS_HI = N - 15 * _ROWS_LO


def _sc_degree(eip):
    mesh = plsc.VectorSubcoreMesh(**_MESH)

    @functools.partial(
        pl.kernel,
        out_type=jax.ShapeDtypeStruct((NC * NS, 1, 2 * N), jnp.float32),
        mesh=mesh,
        compiler_params=_SC_PARAMS,
        scratch_types=[
            pltpu.VMEM((NCH_D, CH), jnp.int32),
            pltpu.VMEM((NCH_D, CH), jnp.int32),
            pltpu.VMEM((1, 2 * N), jnp.float32),
        ],
    )
    def k(ei_hbm, out_hbm, dst2d, src2d, hist):
        c = lax.axis_index("c")
        s = lax.axis_index("s")
        wid = c * NS + s

        pltpu.sync_copy(ei_hbm.at[1, pl.ds(wid * NCH_D, NCH_D)], dst2d)
        pltpu.sync_copy(ei_hbm.at[0, pl.ds(wid * NCH_D, NCH_D)], src2d)

        @pl.loop(0, 2 * N, step=16)
        def _(i):
            hist[0, pl.ds(i, 16)] = jnp.zeros((16,), jnp.float32)

        ones = jnp.ones((16,), jnp.float32)
        hrow = hist.at[0]

        @pl.loop(0, NCH_D)
        def _(j):
            for g in range(CH // 16):
                sl = pl.ds(g * 16, 16)
                plsc.addupdate_scatter(hrow, [dst2d[j, sl] * 2], ones)
                plsc.addupdate_scatter(hrow, [src2d[j, sl] * 2 + 1], ones)

        pltpu.sync_copy(hist, out_hbm.at[wid])

    return k(eip)


def _tc_deg_finish_body(h_ref, o_ref):
    deg = jnp.sum(h_ref[...], axis=0)
    col = lax.broadcasted_iota(jnp.int32, (1, 2 * N), 1)
    deg = deg - jnp.where(col < 2, jnp.float32(PAD), 0.0)
    o_ref[...] = jnp.where(deg > 0, 1.0 / deg, 0.0)


def _tc_deg_finish(hists):
    return pl.pallas_call(
        _tc_deg_finish_body,
        out_shape=jax.ShapeDtypeStruct((1, 2 * N), jnp.float32),
    )(hists)


def _sc_prop(tables, eip, gwp, zeros):
    mesh = plsc.VectorSubcoreMesh(**_MESH)
    KB = 8
    NBLK = NCH_T // KB
    NIT = NBLK // 2

    @functools.partial(
        pl.kernel,
        out_type=jax.ShapeDtypeStruct((2, N, D), jnp.float32),
        mesh=mesh,
        compiler_params=_SC_PARAMS,
        scratch_types=[
            pltpu.VMEM((KB, CH), jnp.int32),
            pltpu.VMEM((KB, CH), jnp.int32),
            pltpu.VMEM((KB, CH), jnp.float32),
            pltpu.VMEM((KB, CH), jnp.int32),
            pltpu.VMEM((KB, CH), jnp.int32),
            pltpu.VMEM((KB, CH), jnp.float32),
            pltpu.VMEM((CH, D), jnp.float32),
            pltpu.VMEM((CH, D), jnp.float32),
            pltpu.VMEM_SHARED((N, D), jnp.float32),
            pltpu.SemaphoreType.DMA,
            pltpu.SemaphoreType.DMA,
            pltpu.SemaphoreType.DMA,
            pltpu.SemaphoreType.DMA,
            pltpu.SemaphoreType.DMA,
            pltpu.SemaphoreType.DMA,
        ],
    )
    def k(tab_hbm, ei_hbm, gw_hbm, z_hbm, s_hbm, gx, sx, wx, gy, sy, wy,
          rows_a, rows_b, accum, gsem_a, gsem_b, ssem_a, ssem_b, xsem, ysem):
        c = lax.axis_index("c")
        s = lax.axis_index("s")
        cb = s * NCH_T
        stx = (gx, sx, wx)
        sty = (gy, sy, wy)

        def fetch(b, st, sem):
            rs = pl.ds(cb + b * KB, KB)
            pltpu.async_copy(ei_hbm.at[c, rs], st[0], sem)
            pltpu.async_copy(ei_hbm.at[1 - c, rs], st[1], sem)
            pltpu.async_copy(gw_hbm.at[c, rs], st[2], sem)

        def wait_fetch(b, st, sem):
            rs = pl.ds(cb + b * KB, KB)
            pltpu.make_async_copy(ei_hbm.at[c, rs], st[0], sem).wait()
            pltpu.make_async_copy(ei_hbm.at[1 - c, rs], st[1], sem).wait()
            pltpu.make_async_copy(gw_hbm.at[c, rs], st[2], sem).wait()

        rs0 = pl.ds(cb, KB)
        pltpu.sync_copy(ei_hbm.at[c, rs0], gx)
        pltpu.sync_copy(ei_hbm.at[1 - c, rs0], sx)
        pltpu.sync_copy(gw_hbm.at[c, rs0], wx)
        fetch(1, sty, ysem)

        rb = s * _ROWS_LO

        @pl.when(s < NS - 1)
        def _():
            pltpu.sync_copy(z_hbm.at[pl.ds(rb, _ROWS_LO)],
                            accum.at[pl.ds(rb, _ROWS_LO)])

        @pl.when(s == NS - 1)
        def _():
            pltpu.sync_copy(z_hbm.at[pl.ds(rb, _ROWS_HI)],
                            accum.at[pl.ds(rb, _ROWS_HI)])

        plsc.subcore_barrier()

        tab_c = tab_hbm.at[c]

        def issue_gather(st, lr, rows, sem):
            pltpu.async_copy(tab_c.at[st[0].at[lr]], rows, sem)

        def wait_gather(st, lr, rows, sem):
            pltpu.make_async_copy(tab_c.at[st[0].at[lr]], rows, sem).wait()

        def scale(st, lr, rows):
            @pl.loop(0, CH, unroll=2)
            def _(r):
                wv = plsc.load_gather(
                    st[2], [jnp.full((16,), lr, jnp.int32),
                            jnp.full((16,), r, jnp.int32)])
                for q in range(D // 16):
                    sl = pl.ds(q * 16, 16)
                    rows[r, sl] = rows[r, sl] * wv

        def issue_scatter(st, lr, rows, sem):
            pltpu.async_copy(rows, accum.at[st[1].at[lr]], sem, add=True)

        def wait_scatter(st, lr, rows, sem):
            pltpu.make_async_copy(rows, accum.at[st[1].at[lr]], sem).wait()

        issue_gather(stx, 0, rows_a, gsem_a)
        issue_gather(stx, 1, rows_b, gsem_b)

        @pl.loop(0, NIT)
        def _(it):
            for p in range(2 * KB // 2):
                cur = stx if p < 4 else sty
                la = 2 * (p % 4)
                lb = la + 1

                wait_gather(cur, la, rows_a, gsem_a)
                scale(cur, la, rows_a)
                issue_scatter(cur, la, rows_a, ssem_a)
                wait_gather(cur, lb, rows_b, gsem_b)
                scale(cur, lb, rows_b)
                issue_scatter(cur, lb, rows_b, ssem_b)

                if p == 3:
                    wait_fetch(2 * it + 1, sty, ysem)

                if p < 7:
                    nxt = stx if p + 1 < 4 else sty
                    nla = 2 * ((p + 1) % 4)
                    wait_scatter(cur, la, rows_a, ssem_a)
                    issue_gather(nxt, nla, rows_a, gsem_a)
                    wait_scatter(cur, lb, rows_b, ssem_b)
                    issue_gather(nxt, nla + 1, rows_b, gsem_b)

                if p == 3:
                    @pl.when(it < NIT - 1)
                    def _():
                        fetch(2 * it + 2, stx, xsem)

                if p == 7:
                    @pl.when(it < NIT - 1)
                    def _():
                        wait_fetch(2 * it + 2, stx, xsem)
                        wait_scatter(cur, la, rows_a, ssem_a)
                        issue_gather(stx, 0, rows_a, gsem_a)
                        wait_scatter(cur, lb, rows_b, ssem_b)
                        issue_gather(stx, 1, rows_b, gsem_b)
                        fetch(2 * it + 3, sty, ysem)

        wait_scatter(sty, 6, rows_a, ssem_a)
        wait_scatter(sty, 7, rows_b, ssem_b)

        plsc.subcore_barrier()

        @pl.when(s < NS - 1)
        def _():
            pltpu.sync_copy(accum.at[pl.ds(rb, _ROWS_LO)],
                            s_hbm.at[c].at[pl.ds(rb, _ROWS_LO)])

        @pl.when(s == NS - 1)
        def _():
            pltpu.sync_copy(accum.at[pl.ds(rb, _ROWS_HI)],
                            s_hbm.at[c].at[pl.ds(rb, _ROWS_HI)])

    return k(tables, eip, gwp, zeros)


def _sc_pair_gather(h, idx):
    mesh = plsc.VectorSubcoreMesh(**_MESH)
    B = idx.shape[0]
    per_tile = B // (NC * NS)

    @functools.partial(
        pl.kernel,
        out_type=jax.ShapeDtypeStruct((B, 3 * D), jnp.float32),
        mesh=mesh,
        compiler_params=_SC_PARAMS,
        scratch_types=[
            pltpu.VMEM((B,), jnp.int32),
            pltpu.VMEM((per_tile, 3 * D), jnp.float32),
            pltpu.SemaphoreType.DMA,
        ],
    )
    def k(h_hbm, i_hbm, o_hbm, idx_v, rows, sem):
        c = lax.axis_index("c")
        s = lax.axis_index("s")
        wid = c * NS + s
        base = wid * per_tile
        pltpu.sync_copy(i_hbm, idx_v)
        pltpu.async_copy(h_hbm.at[idx_v.at[pl.ds(base, per_tile)]], rows,
                         sem).wait()
        pltpu.sync_copy(rows, o_hbm.at[pl.ds(base, per_tile)])

    return k(h, idx)


_RB = 2000


def _tc_proj_body(h_ref, wu_ref, wd_ref, wb_ref, tab_ref, bias_ref):
    hb = h_ref[...]
    tab_ref[0] = jnp.dot(hb, wu_ref[...], preferred_element_type=jnp.float32)
    tab_ref[1] = jnp.dot(hb, wd_ref[...], preferred_element_type=jnp.float32)
    bias_ref[...] = jnp.dot(hb, wb_ref[...],
                            preferred_element_type=jnp.float32)


def _tc_proj(h, wu, wd, wb):
    din = h.shape[1]
    return pl.pallas_call(
        _tc_proj_body,
        grid=(N // _RB,),
        in_specs=[
            pl.BlockSpec((_RB, din), lambda i: (i, 0)),
            pl.BlockSpec((din, D), lambda i: (0, 0)),
            pl.BlockSpec((din, D), lambda i: (0, 0)),
            pl.BlockSpec((din, D), lambda i: (0, 0)),
        ],
        out_specs=[
            pl.BlockSpec((2, _RB, D), lambda i: (0, i, 0)),
            pl.BlockSpec((_RB, D), lambda i: (i, 0)),
        ],
        out_shape=[
            jax.ShapeDtypeStruct((2, N, D), jnp.float32),
            jax.ShapeDtypeStruct((N, D), jnp.float32),
        ],
    )(h, wu, wd, wb)


def _tc_finish_body(s_ref, dinv_ref, bias_ref, h_ref):
    d = dinv_ref[...]
    xu = s_ref[0] * d[:, 0:1]
    xd = s_ref[1] * d[:, 1:2]
    bx = bias_ref[...]
    ss = (jnp.sum(xu * xu, axis=1) + jnp.sum(xd * xd, axis=1)
          + jnp.sum(bx * bx, axis=1))
    inv = 1.0 / jnp.maximum(jnp.sqrt(ss), 1e-12)

    def leaky(t):
        return jnp.where(t >= 0, t, 0.1 * t)

    h_ref[:, 0:D] = leaky(xu * inv[:, None])
    h_ref[:, D:2 * D] = leaky(xd * inv[:, None])
    h_ref[:, 2 * D:3 * D] = leaky(bx * inv[:, None])


def _tc_finish(S, deg_inv, bias):
    return pl.pallas_call(
        _tc_finish_body,
        grid=(N // _RB,),
        in_specs=[
            pl.BlockSpec((2, _RB, D), lambda i: (0, i, 0)),
            pl.BlockSpec((_RB, 2), lambda i: (i, 0)),
            pl.BlockSpec((_RB, D), lambda i: (i, 0)),
        ],
        out_specs=pl.BlockSpec((_RB, 3 * D), lambda i: (i, 0)),
        out_shape=jax.ShapeDtypeStruct((N, 3 * D), jnp.float32),
    )(S, deg_inv, bias)


def _tc_decoder_body(a_ref, b_ref, p1_ref, p2_ref, o_ref):
    a = a_ref[...]
    p1 = p1_ref[...]
    t1 = jnp.dot(a, p1, preferred_element_type=jnp.float32)
    t2 = jnp.dot(t1, p2_ref[...], preferred_element_type=jnp.float32)
    p = lax.dot_general(t2, p1, (((1,), (1,)), ((), ())),
                        preferred_element_type=jnp.float32)
    o_ref[...] = jnp.sum(p * b_ref[...], axis=1, keepdims=True)


def _tc_decoder(a, b, p1, p2):
    npairs = a.shape[0]
    return pl.pallas_call(
        _tc_decoder_body,
        out_shape=jax.ShapeDtypeStruct((npairs, 1), jnp.float32),
    )(a, b, p1, p2)


def kernel(x, edge_index, drug_index, label, W1_up, W1_down, W1_bias, g1_up,
           g1_down, W2_up, W2_down, W2_bias, g2_up, g2_down, W3_up, W3_down,
           W3_bias, g3_up, g3_down, P1, P2):
    ei = edge_index.astype(jnp.int32)
    eip = jnp.concatenate(
        [ei, jnp.zeros((2, PAD), jnp.int32)], axis=1).reshape(2, NCHP, CH)
    zeros = jnp.zeros((N, D), jnp.float32)
    ones_d = jnp.ones((ND,), jnp.float32)
    zeros_p = jnp.zeros((PAD,), jnp.float32)

    hists = _sc_degree(eip)
    deg_inv = _tc_deg_finish(hists).reshape(N, 2)

    h = x
    for wu, wd, wb, gu, gd in (
        (W1_up, W1_down, W1_bias, g1_up, g1_down),
        (W2_up, W2_down, W2_bias, g2_up, g2_down),
        (W3_up, W3_down, W3_bias, g3_up, g3_down),
    ):
        tabs, bias = _tc_proj(h, wu, wd, wb)
        gwp = jnp.stack([jnp.concatenate([gu, ones_d, zeros_p]),
                         jnp.concatenate([gd, ones_d, zeros_p])]
                        ).reshape(2, NCHP, CH)
        S = _sc_prop(tabs, eip, gwp, zeros)
        h = _tc_finish(S, deg_inv, bias)

    di = drug_index.reshape(-1, 2).astype(jnp.int32)
    ia = di[:, 0] - 1
    ib = di[:, 1] - 1
    ia = jnp.where(ia < 0, ia + N, ia)
    ib = jnp.where(ib < 0, ib + N, ib)
    ab = _sc_pair_gather(h, jnp.concatenate([ia, ib]))
    a = ab[: di.shape[0]]
    b = ab[di.shape[0]:]
    return _tc_decoder(a, b, P1, P2)
